# Initial kernel scaffold; baseline (speedup 1.0000x reference)
import jax
import jax.numpy as jnp
from jax.experimental import pallas as pl


def kernel(x, table):
    raise NotImplementedError("write your pallas kernel here")


# speedup vs baseline: 1.1026x; 1.1026x over previous
import functools

import jax
import jax.numpy as jnp
from jax import lax
from jax.experimental import pallas as pl
from jax.experimental.pallas import tpu as pltpu
from jax.experimental.pallas import tpu_sc as plsc

_DIM = 32
_NC = 2
_NS = 16
_NW = _NC * _NS


def _make_gather(n_idx, dim):
    b_per_w = n_idx // _NW
    chunk = 1024
    nsteps = b_per_w // chunk
    mesh = plsc.VectorSubcoreMesh(core_axis_name="c", subcore_axis_name="s")

    @functools.partial(
        pl.kernel,
        out_type=jax.ShapeDtypeStruct((n_idx, dim), jnp.float32),
        mesh=mesh,
        scratch_types=[
            pltpu.VMEM((b_per_w,), jnp.int32),
            pltpu.VMEM((chunk, dim), jnp.float32),
            pltpu.SemaphoreType.DMA,
        ],
        compiler_params=pltpu.CompilerParams(use_tc_tiling_on_sc=False),
    )
    def gather_kernel(idx_hbm, table_hbm, out_hbm, idx_v, rows_v, sem):
        wid = lax.axis_index("s") * _NC + lax.axis_index("c")
        base = wid * b_per_w
        pltpu.sync_copy(idx_hbm.at[pl.ds(base, b_per_w)], idx_v)

        def step(j, carry):
            off = j * chunk
            pltpu.async_copy(
                table_hbm.at[idx_v.at[pl.ds(off, chunk)]], rows_v, sem
            ).wait()
            pltpu.sync_copy(rows_v, out_hbm.at[pl.ds(base + off, chunk)])
            return carry

        lax.fori_loop(0, nsteps, step, 0)

    return gather_kernel


@jax.jit
def kernel(x, table):
    b, h = x.shape
    flat = x.reshape(b * h)
    out = _make_gather(b * h, table.shape[1])(flat, table)
    return out.reshape(b, h, _DIM)


# speedup vs baseline: 1.1108x; 1.0074x over previous
import functools

import jax
import jax.numpy as jnp
from jax import lax
from jax.experimental import pallas as pl
from jax.experimental.pallas import tpu as pltpu
from jax.experimental.pallas import tpu_sc as plsc

_DIM = 32
_NC = 2
_NS = 16
_NW = _NC * _NS


def _make_gather(n_idx, dim):
    b_per_w = n_idx // _NW
    chunk = 1280
    nsteps = b_per_w // chunk
    mesh = plsc.VectorSubcoreMesh(core_axis_name="c", subcore_axis_name="s")

    @functools.partial(
        pl.kernel,
        out_type=jax.ShapeDtypeStruct((n_idx, dim), jnp.float32),
        mesh=mesh,
        scratch_types=[
            pltpu.VMEM((b_per_w,), jnp.int32),
            pltpu.VMEM((chunk, dim), jnp.float32),
            pltpu.VMEM((chunk, dim), jnp.float32),
            pltpu.SemaphoreType.DMA,
            pltpu.SemaphoreType.DMA,
            pltpu.SemaphoreType.DMA,
            pltpu.SemaphoreType.DMA,
        ],
        compiler_params=pltpu.CompilerParams(use_tc_tiling_on_sc=False),
    )
    def gather_kernel(idx_hbm, table_hbm, out_hbm, idx_v, rows0, rows1,
                      g0, g1, s0, s1):
        wid = lax.axis_index("s") * _NC + lax.axis_index("c")
        base = wid * b_per_w
        pltpu.sync_copy(idx_hbm.at[pl.ds(base, b_per_w)], idx_v)

        rows = (rows0, rows1)
        gsem = (g0, g1)
        ssem = (s0, s1)

        def gather_desc(j, b):
            return pltpu.make_async_copy(
                table_hbm.at[idx_v.at[pl.ds(j * chunk, chunk)]],
                rows[b], gsem[b])

        def store_desc(j, b):
            return pltpu.make_async_copy(
                rows[b], out_hbm.at[pl.ds(base + j * chunk, chunk)], ssem[b])

        gather_desc(0, 0).start()
        for j in range(nsteps):
            b = j % 2
            gather_desc(j, b).wait()
            if j >= 1:
                store_desc(j - 1, 1 - b).wait()
            if j + 1 < nsteps:
                gather_desc(j + 1, 1 - b).start()
            store_desc(j, b).start()
        store_desc(nsteps - 1, (nsteps - 1) % 2).wait()

    return gather_kernel


@jax.jit
def kernel(x, table):
    b, h = x.shape
    flat = x.reshape(b * h)
    out = _make_gather(b * h, table.shape[1])(flat, table)
    return out.reshape(b, h, _DIM)


# speedup vs baseline: 1.8033x; 1.6234x over previous
import functools

import jax
import jax.numpy as jnp
from jax import lax
from jax.experimental import pallas as pl
from jax.experimental.pallas import tpu as pltpu
from jax.experimental.pallas import tpu_sc as plsc

_NC = 2
_NS = 16
_NW = _NC * _NS
_CR = 16


def _make_embed(n_rows, n_hist, dim):
    rows_per_w = n_rows // _NW
    nsteps = rows_per_w // _CR
    mesh = plsc.VectorSubcoreMesh(core_axis_name="c", subcore_axis_name="s")

    @functools.partial(
        pl.kernel,
        out_type=jax.ShapeDtypeStruct((n_rows, n_hist, dim), jnp.float32),
        mesh=mesh,
        scratch_types=[
            pltpu.VMEM((_CR, n_hist), jnp.int32),
            pltpu.VMEM((_CR, n_hist), jnp.int32),
            pltpu.VMEM((_CR, n_hist, dim), jnp.float32),
            pltpu.VMEM((_CR, n_hist, dim), jnp.float32),
            pltpu.SemaphoreType.DMA,
            pltpu.SemaphoreType.DMA,
            pltpu.SemaphoreType.DMA,
            pltpu.SemaphoreType.DMA,
            pltpu.SemaphoreType.DMA,
            pltpu.SemaphoreType.DMA,
        ],
        compiler_params=pltpu.CompilerParams(use_tc_tiling_on_sc=False),
    )
    def embed_kernel(x_hbm, table_hbm, out_hbm, idx0, idx1, rows0, rows1,
                     i0, i1, g0, g1, s0, s1):
        wid = lax.axis_index("s") * _NC + lax.axis_index("c")
        row0 = wid * rows_per_w

        idx = (idx0, idx1)
        rows = (rows0, rows1)
        isem = (i0, i1)
        gsem = (g0, g1)
        ssem = (s0, s1)

        def idx_desc(j, b):
            return pltpu.make_async_copy(
                x_hbm.at[pl.ds(row0 + j * _CR, _CR), :], idx[b], isem[b])

        def start_gathers(b):
            for i in range(_CR):
                pltpu.make_async_copy(
                    table_hbm.at[idx[b].at[i, :]], rows[b].at[i], gsem[b]
                ).start()

        def drain_gathers(b):
            pltpu.make_async_copy(
                out_hbm.at[pl.ds(0, _CR), :, :], rows[b], gsem[b]).wait()

        def store_desc(j, b):
            return pltpu.make_async_copy(
                rows[b], out_hbm.at[pl.ds(row0 + j * _CR, _CR), :, :],
                ssem[b])

        idx_desc(0, 0).start()
        for j in range(nsteps):
            b = j % 2
            if j >= 2:
                store_desc(j - 2, b).wait()
            idx_desc(j, b).wait()
            start_gathers(b)
            if j >= 1:
                drain_gathers(1 - b)
                store_desc(j - 1, 1 - b).start()
            if j + 1 < nsteps:
                idx_desc(j + 1, 1 - b).start()
        bl = (nsteps - 1) % 2
        drain_gathers(bl)
        store_desc(nsteps - 1, bl).start()
        store_desc(nsteps - 2, 1 - bl).wait()
        store_desc(nsteps - 1, bl).wait()

    return embed_kernel


def kernel(x, table):
    return _make_embed(x.shape[0], x.shape[1], table.shape[1])(x, table)
